# baseline (speedup 1.0000x reference)
import functools

import jax
import jax.numpy as jnp
from jax import lax
from jax.experimental import pallas as pl
from jax.experimental.pallas import tpu as pltpu
from jax.experimental.pallas import tpu_sc as plsc

H = 768
E = 8
FF = 4 * H
T = 2048
NEG = -1e30

BLK = 512
BLK_LOG2 = 9
SNB = 16
S = SNB * BLK
NW = 32
SPT = S // NW
TPT = T // NW
GROUPS = T // 16
GPT = TPT // 16

NJ = 4
FBLK = FF // NJ


def _gating_body(x_ref, gw_ref, gb_ref, eidx_ref, wv_ref, laux_ref, cnt_ref):
    x = x_ref[...]
    gw = gw_ref[...]
    gb = gb_ref[...]
    lt = lax.dot_general(gw, x, (((1,), (1,)), ((), ())),
                         preferred_element_type=jnp.float32) + gb
    rows = lax.broadcasted_iota(jnp.int32, (E, T), 0)
    m1 = jnp.max(lt, axis=0, keepdims=True)
    e1 = jnp.min(jnp.where(lt == m1, rows, E), axis=0, keepdims=True)
    mask1 = (rows == e1)
    lt2 = jnp.where(mask1, NEG, lt)
    m2 = jnp.max(lt2, axis=0, keepdims=True)
    e2 = jnp.min(jnp.where(lt2 == m2, rows, E), axis=0, keepdims=True)
    mask2 = (rows == e2)
    w1 = 1.0 / (1.0 + jnp.exp(m2 - m1))
    w2 = 1.0 - w1
    eidx_ref[...] = jnp.concatenate([e1, e2], axis=0)
    wv_ref[...] = jnp.concatenate([w1, w2], axis=0)
    cnt_ref[...] = jnp.sum(mask1.astype(jnp.float32) + mask2.astype(jnp.float32),
                           axis=1, keepdims=True)
    p = jnp.exp(lt - m1)
    p = p / jnp.sum(p, axis=0, keepdims=True)
    pm = jnp.mean(p, axis=1, keepdims=True)
    laux_ref[0, 0] = jnp.sum(pm * pm) * E


def _gating(x2d, gate_w, gate_b):
    return pl.pallas_call(
        _gating_body,
        out_shape=(
            jax.ShapeDtypeStruct((2, T), jnp.int32),
            jax.ShapeDtypeStruct((2, T), jnp.float32),
            jax.ShapeDtypeStruct((1, 1), jnp.float32),
            jax.ShapeDtypeStruct((E, 1), jnp.float32),
        ),
        out_specs=(
            pl.BlockSpec(memory_space=pltpu.VMEM),
            pl.BlockSpec(memory_space=pltpu.VMEM),
            pl.BlockSpec(memory_space=pltpu.SMEM),
            pl.BlockSpec(memory_space=pltpu.VMEM),
        ),
    )(x2d, gate_w, gate_b.reshape(E, 1))


_MESH = dict(core_axis_name="c", subcore_axis_name="s")


@functools.partial(
    pl.kernel,
    out_type=(
        jax.ShapeDtypeStruct((S, H), jnp.float32),
        jax.ShapeDtypeStruct((S, 1), jnp.float32),
        jax.ShapeDtypeStruct((16,), jnp.int32),
        jax.ShapeDtypeStruct((16,), jnp.int32),
        jax.ShapeDtypeStruct((8,), jnp.int32),
        jax.ShapeDtypeStruct((NW, 2, TPT), jnp.int32),
    ),
    mesh=plsc.VectorSubcoreMesh(**_MESH),
    compiler_params=pltpu.CompilerParams(needs_layout_passes=False),
    scratch_types=[
        pltpu.VMEM((2, T), jnp.int32),
        pltpu.VMEM((2, T), jnp.float32),
        pltpu.VMEM((SPT,), jnp.int32),
        pltpu.VMEM((SPT, 1), jnp.float32),
        pltpu.VMEM((64, H), jnp.float32),
        pltpu.VMEM((2, TPT), jnp.int32),
        pltpu.VMEM((16,), jnp.int32),
        pltpu.VMEM((16,), jnp.int32),
        pltpu.SemaphoreType.DMA,
    ],
)
def _route(eidx_hbm, wv_hbm, x_hbm,
           xs_hbm, sw_hbm, wblk_hbm, xblk_hbm, nreal_hbm, cidx_hbm,
           eidx_v, wv_v, stok_v, sw_v, rows_v, cpos_v, stg1_v, stg2_v, sem):
    wid = lax.axis_index("s") * 2 + lax.axis_index("c")
    base = wid * SPT
    g0 = wid * GPT
    lanes = lax.iota(jnp.int32, 16)
    zi = jnp.zeros((16,), jnp.int32)

    pltpu.sync_copy(eidx_hbm, eidx_v)
    pltpu.sync_copy(wv_hbm, wv_v)

    zs = jnp.int32(0)

    def pass_a(g, cnts):
        e1 = eidx_v[0, pl.ds(g * 16, 16)]
        e2 = eidx_v[1, pl.ds(g * 16, 16)]
        return tuple(
            cnts[e] + jnp.sum(
                jnp.logical_or(e1 == e, e2 == e).astype(jnp.int32))
            for e in range(E))

    cnts = lax.fori_loop(0, GROUPS, pass_a, (zs,) * E)

    offs = []
    acc = zs
    for e in range(E):
        offs.append(acc)
        acc = acc + (((cnts[e] + (BLK - 1)) >> BLK_LOG2) << BLK_LOG2)
    nreal = acc >> BLK_LOG2

    xb = jnp.minimum(lanes, nreal - 1)
    wb = zi
    for e in range(1, E):
        wb = wb + jnp.where(xb * BLK >= offs[e], 1, 0)

    @pl.when(wid == 0)
    def _():
        stg1_v[...] = wb
        pltpu.sync_copy(stg1_v, wblk_hbm)
        stg2_v[...] = xb
        pltpu.sync_copy(stg2_v, xblk_hbm)
        stg2_v[...] = zi + nreal
        pltpu.sync_copy(stg2_v.at[pl.ds(0, 8)], nreal_hbm)

    zf = jnp.zeros((16,), jnp.float32)
    for c in range(SPT // 16):
        stok_v[pl.ds(c * 16, 16)] = zi
        plsc.store_scatter(sw_v, [lanes + c * 16, zi], zf)

    def pass_b(g, runs):
        e1 = eidx_v[0, pl.ds(g * 16, 16)]
        e2 = eidx_v[1, pl.ds(g * 16, 16)]
        w1 = wv_v[0, pl.ds(g * 16, 16)]
        w2 = wv_v[1, pl.ds(g * 16, 16)]
        p0 = zi
        p1 = zi
        new_runs = []
        for e in range(E):
            m1 = e1 == e
            m2 = e2 == e
            ohb = jnp.logical_or(m1, m2)
            oh = ohb.astype(jnp.int32)
            pref = plsc.cumsum(oh) - oh
            pos_e = (offs[e] + runs[e]) + pref
            p0 = jnp.where(m1, pos_e, p0)
            p1 = jnp.where(m2, pos_e, p1)
            new_runs.append(runs[e] + jnp.sum(oh))
        toks = g * 16 + lanes
        in0 = jnp.logical_and(p0 >= base, p0 < base + SPT)
        in1 = jnp.logical_and(p1 >= base, p1 < base + SPT)
        plsc.store_scatter(stok_v, [p0 - base], toks, mask=in0)
        plsc.store_scatter(stok_v, [p1 - base], toks, mask=in1)
        plsc.store_scatter(sw_v, [p0 - base, zi], w1, mask=in0)
        plsc.store_scatter(sw_v, [p1 - base, zi], w2, mask=in1)

        @pl.when(jnp.logical_and(g >= g0, g < g0 + GPT))
        def _():
            cpos_v[0, pl.ds((g - g0) * 16, 16)] = p0
            cpos_v[1, pl.ds((g - g0) * 16, 16)] = p1

        return tuple(new_runs)

    lax.fori_loop(0, GROUPS, pass_b, (zs,) * E)

    pltpu.sync_copy(cpos_v, cidx_hbm.at[wid])
    pltpu.sync_copy(sw_v, sw_hbm.at[pl.ds(base, SPT), :])

    for c in range(SPT // 64):
        idx_ref = stok_v.at[pl.ds(c * 64, 64)]
        pltpu.async_copy(x_hbm.at[idx_ref], rows_v, sem).wait()
        pltpu.sync_copy(rows_v, xs_hbm.at[pl.ds(base + c * 64, 64), :])


def _gffn_body(wblk_s, xblk_s, nreal_s,
               x_ref, w1_ref, b1_ref, w2_ref, b2_ref, sw_ref, out_ref):
    j = pl.program_id(1)

    @pl.when(pl.program_id(0) < nreal_s[0])
    def _():
        xb = x_ref[...]
        h = lax.dot_general(xb, w1_ref[0], (((1,), (1,)), ((), ())),
                            preferred_element_type=jnp.float32)
        h = h + b1_ref[0]
        h = 0.5 * h * (1.0 + lax.erf(h * 0.7071067811865476))
        part = lax.dot_general(h, w2_ref[0], (((1,), (1,)), ((), ())),
                               preferred_element_type=jnp.float32)
        prev = jnp.where(j == 0, 0.0, out_ref[...])
        acc = prev + part
        acc = jnp.where(j == NJ - 1, (acc + b2_ref[0]) * sw_ref[...], acc)
        out_ref[...] = acc


def _gffn(wblk, xblk, nreal, xs, W1, b1, W2, b2, sw):
    grid_spec = pltpu.PrefetchScalarGridSpec(
        num_scalar_prefetch=3,
        grid=(SNB, NJ),
        in_specs=[
            pl.BlockSpec((BLK, H), lambda i, j, wb, xb, nr: (xb[i], 0)),
            pl.BlockSpec((1, FBLK, H), lambda i, j, wb, xb, nr: (wb[i], j, 0)),
            pl.BlockSpec((1, 1, FBLK), lambda i, j, wb, xb, nr: (wb[i], 0, j)),
            pl.BlockSpec((1, H, FBLK), lambda i, j, wb, xb, nr: (wb[i], 0, j)),
            pl.BlockSpec((1, 1, H), lambda i, j, wb, xb, nr: (wb[i], 0, 0)),
            pl.BlockSpec((BLK, 1), lambda i, j, wb, xb, nr: (xb[i], 0)),
        ],
        out_specs=pl.BlockSpec((BLK, H), lambda i, j, wb, xb, nr: (xb[i], 0)),
    )
    return pl.pallas_call(
        _gffn_body,
        grid_spec=grid_spec,
        out_shape=jax.ShapeDtypeStruct((S, H), jnp.float32),
    )(wblk, xblk, nreal, xs, W1, b1.reshape(E, 1, FF), W2,
      b2.reshape(E, 1, H), sw)


@functools.partial(
    pl.kernel,
    out_type=jax.ShapeDtypeStruct((T, H), jnp.float32),
    mesh=plsc.VectorSubcoreMesh(**_MESH),
    compiler_params=pltpu.CompilerParams(needs_layout_passes=False),
    scratch_types=[
        pltpu.VMEM((TPT,), jnp.int32),
        pltpu.VMEM((TPT,), jnp.int32),
        pltpu.VMEM((TPT, H), jnp.float32),
        pltpu.VMEM((TPT, H), jnp.float32),
        pltpu.SemaphoreType.DMA,
        pltpu.SemaphoreType.DMA,
    ],
)
def _combine(ffn_hbm, cidx_hbm, out_hbm, i0_v, i1_v, r0_v, r1_v, sem0, sem1):
    wid = lax.axis_index("s") * 2 + lax.axis_index("c")
    tb = wid * TPT
    pltpu.sync_copy(cidx_hbm.at[wid, 0], i0_v)
    pltpu.sync_copy(cidx_hbm.at[wid, 1], i1_v)
    d0 = pltpu.async_copy(ffn_hbm.at[i0_v], r0_v, sem0)
    d1 = pltpu.async_copy(ffn_hbm.at[i1_v], r1_v, sem1)
    d0.wait()
    d1.wait()

    def add_row(r, carry):
        for c in range(H // 16):
            sl = pl.ds(c * 16, 16)
            r0_v[r, sl] = r0_v[r, sl] + r1_v[r, sl]
        return carry

    lax.fori_loop(0, TPT, add_row, 0)
    pltpu.sync_copy(r0_v, out_hbm.at[pl.ds(tb, TPT), :])


def kernel(x, gate_w, gate_b, W1, b1, W2, b2):
    bsz, seq, hidden = x.shape
    x2d = x.reshape(T, H)
    eidx, wvals, laux, counts = _gating(x2d, gate_w, gate_b)
    xs, sw, wblk, xblk, nreal, cidx = _route(eidx, wvals, x2d)
    ffn_out = _gffn(wblk, xblk, nreal, xs, W1, b1, W2, b2, sw)
    out2d = _combine(ffn_out, cidx)
    return out2d.reshape(bsz, seq, hidden), laux[0, 0], counts.reshape(E)


# speedup vs baseline: 2.0908x; 2.0908x over previous
import functools

import jax
import jax.numpy as jnp
from jax import lax
from jax.experimental import pallas as pl
from jax.experimental.pallas import tpu as pltpu
from jax.experimental.pallas import tpu_sc as plsc

H = 768
E = 8
FF = 4 * H
T = 2048
NEG = -1e30

BLK = 512
BLK_LOG2 = 9
SNB = 16
S = SNB * BLK
NW = 32
TPT = T // NW

NJ = 4
FBLK = FF // NJ


def _gating_body(x_ref, gw_ref, gb_ref,
                 cidx_ref, wv_ref, meta_ref, laux_ref, cnt_ref):
    x = x_ref[...]
    gw = gw_ref[...]
    gb = gb_ref[...]
    lt = lax.dot_general(gw, x, (((1,), (1,)), ((), ())),
                         preferred_element_type=jnp.float32) + gb
    rows = lax.broadcasted_iota(jnp.int32, (E, T), 0)
    m1 = jnp.max(lt, axis=0, keepdims=True)
    e1 = jnp.min(jnp.where(lt == m1, rows, E), axis=0, keepdims=True)
    mask1 = (rows == e1)
    lt2 = jnp.where(mask1, NEG, lt)
    m2 = jnp.max(lt2, axis=0, keepdims=True)
    e2 = jnp.min(jnp.where(lt2 == m2, rows, E), axis=0, keepdims=True)
    mask2 = (rows == e2)
    w1 = 1.0 / (1.0 + jnp.exp(m2 - m1))
    w2 = 1.0 - w1
    wv_ref[...] = jnp.concatenate([w1, w2], axis=0)

    p = jnp.exp(lt - m1)
    p = p / jnp.sum(p, axis=0, keepdims=True)
    pm = jnp.mean(p, axis=1, keepdims=True)
    laux_ref[0, 0] = jnp.sum(pm * pm) * E

    oh = jnp.where(jnp.logical_or(mask1, mask2), 1.0, 0.0)
    pre = oh
    k = 1
    while k < T:
        shifted = jnp.concatenate(
            [jnp.zeros((E, k), jnp.float32), pre[:, :T - k]], axis=1)
        pre = pre + shifted
        k *= 2
    ranks = pre - oh

    cnt = jnp.sum(oh, axis=1, keepdims=True)
    cnt_ref[...] = cnt

    cnt_i = cnt.astype(jnp.int32)
    padded = ((cnt_i + (BLK - 1)) >> BLK_LOG2) << BLK_LOG2
    off = padded
    k = 1
    while k < E:
        off = off + jnp.concatenate(
            [jnp.zeros((k, 1), jnp.int32), off[:E - k]], axis=0)
        k *= 2
    offs = off - padded
    total = jnp.sum(padded, axis=0, keepdims=True)
    nreal = total >> BLK_LOG2

    pos = offs.astype(jnp.float32) + ranks
    pos0 = jnp.sum(jnp.where(mask1, pos, 0.0), axis=0, keepdims=True)
    pos1 = jnp.sum(jnp.where(mask2, pos, 0.0), axis=0, keepdims=True)
    cidx_ref[...] = jnp.concatenate([pos0, pos1], axis=0).astype(jnp.int32)

    lanes = lax.broadcasted_iota(jnp.int32, (1, SNB), 1)
    xb = jnp.minimum(lanes, nreal - 1)
    cmp = (xb * BLK >= offs).astype(jnp.int32)
    wb = jnp.sum(cmp, axis=0, keepdims=True) - 1
    nr = nreal + jnp.zeros((1, SNB), jnp.int32)
    meta_ref[...] = jnp.concatenate([xb, wb, nr], axis=0)


def _gating(x2d, gate_w, gate_b):
    return pl.pallas_call(
        _gating_body,
        out_shape=(
            jax.ShapeDtypeStruct((2, T), jnp.int32),
            jax.ShapeDtypeStruct((2, T), jnp.float32),
            jax.ShapeDtypeStruct((3, SNB), jnp.int32),
            jax.ShapeDtypeStruct((1, 1), jnp.float32),
            jax.ShapeDtypeStruct((E, 1), jnp.float32),
        ),
        out_specs=(
            pl.BlockSpec(memory_space=pltpu.VMEM),
            pl.BlockSpec(memory_space=pltpu.VMEM),
            pl.BlockSpec(memory_space=pltpu.VMEM),
            pl.BlockSpec(memory_space=pltpu.SMEM),
            pl.BlockSpec(memory_space=pltpu.VMEM),
        ),
    )(x2d, gate_w, gate_b.reshape(E, 1))


_MESH = dict(core_axis_name="c", subcore_axis_name="s")


@functools.partial(
    pl.kernel,
    out_type=jax.ShapeDtypeStruct((S, H), jnp.float32),
    mesh=plsc.VectorSubcoreMesh(**_MESH),
    compiler_params=pltpu.CompilerParams(needs_layout_passes=False),
    scratch_types=[
        pltpu.VMEM((TPT,), jnp.int32),
        pltpu.VMEM((TPT,), jnp.int32),
        pltpu.VMEM((TPT, H), jnp.float32),
        pltpu.SemaphoreType.DMA,
        pltpu.SemaphoreType.DMA,
    ],
)
def _dispatch(x_hbm, cidx_hbm, xs_hbm, i0_v, i1_v, rows_v, s0, s1):
    wid = lax.axis_index("s") * 2 + lax.axis_index("c")
    tb = wid * TPT
    pltpu.sync_copy(cidx_hbm.at[wid, 0], i0_v)
    pltpu.sync_copy(cidx_hbm.at[wid, 1], i1_v)
    pltpu.sync_copy(x_hbm.at[pl.ds(tb, TPT), :], rows_v)
    d0 = pltpu.async_copy(rows_v, xs_hbm.at[i0_v], s0)
    d1 = pltpu.async_copy(rows_v, xs_hbm.at[i1_v], s1)
    d0.wait()
    d1.wait()


def _gffn_body(xblk_s, wblk_s, nreal_s,
               x_ref, w1_ref, b1_ref, w2_ref, b2_ref, out_ref):
    j = pl.program_id(1)

    @pl.when(pl.program_id(0) < nreal_s[0])
    def _():
        xb = x_ref[...]
        h = lax.dot_general(xb, w1_ref[0], (((1,), (1,)), ((), ())),
                            preferred_element_type=jnp.float32)
        h = h + b1_ref[0]
        h = 0.5 * h * (1.0 + lax.erf(h * 0.7071067811865476))
        part = lax.dot_general(h, w2_ref[0], (((1,), (1,)), ((), ())),
                               preferred_element_type=jnp.float32)
        prev = jnp.where(j == 0, 0.0, out_ref[...])
        acc = prev + part
        acc = jnp.where(j == NJ - 1, acc + b2_ref[0], acc)
        out_ref[...] = acc


def _gffn(xblk, wblk, nreal, xs, W1, b1, W2, b2):
    grid_spec = pltpu.PrefetchScalarGridSpec(
        num_scalar_prefetch=3,
        grid=(SNB, NJ),
        in_specs=[
            pl.BlockSpec((BLK, H), lambda i, j, xb, wb, nr: (xb[i], 0)),
            pl.BlockSpec((1, FBLK, H), lambda i, j, xb, wb, nr: (wb[i], j, 0)),
            pl.BlockSpec((1, 1, FBLK), lambda i, j, xb, wb, nr: (wb[i], 0, j)),
            pl.BlockSpec((1, H, FBLK), lambda i, j, xb, wb, nr: (wb[i], 0, j)),
            pl.BlockSpec((1, 1, H), lambda i, j, xb, wb, nr: (wb[i], 0, 0)),
        ],
        out_specs=pl.BlockSpec((BLK, H), lambda i, j, xb, wb, nr: (xb[i], 0)),
    )
    return pl.pallas_call(
        _gffn_body,
        grid_spec=grid_spec,
        out_shape=jax.ShapeDtypeStruct((S, H), jnp.float32),
    )(xblk, wblk, nreal, xs, W1, b1.reshape(E, 1, FF), W2,
      b2.reshape(E, 1, H))


@functools.partial(
    pl.kernel,
    out_type=jax.ShapeDtypeStruct((T, H), jnp.float32),
    mesh=plsc.VectorSubcoreMesh(**_MESH),
    compiler_params=pltpu.CompilerParams(needs_layout_passes=False),
    scratch_types=[
        pltpu.VMEM((TPT,), jnp.int32),
        pltpu.VMEM((TPT,), jnp.int32),
        pltpu.VMEM((TPT, H), jnp.float32),
        pltpu.VMEM((TPT, H), jnp.float32),
        pltpu.VMEM((TPT,), jnp.float32),
        pltpu.VMEM((TPT,), jnp.float32),
        pltpu.SemaphoreType.DMA,
        pltpu.SemaphoreType.DMA,
    ],
)
def _combine(ffn_hbm, cidx_hbm, wv_hbm, out_hbm,
             i0_v, i1_v, r0_v, r1_v, w0_v, w1_v, sem0, sem1):
    wid = lax.axis_index("s") * 2 + lax.axis_index("c")
    tb = wid * TPT
    pltpu.sync_copy(cidx_hbm.at[wid, 0], i0_v)
    pltpu.sync_copy(cidx_hbm.at[wid, 1], i1_v)
    pltpu.sync_copy(wv_hbm.at[wid, 0], w0_v)
    pltpu.sync_copy(wv_hbm.at[wid, 1], w1_v)
    d0 = pltpu.async_copy(ffn_hbm.at[i0_v], r0_v, sem0)
    d1 = pltpu.async_copy(ffn_hbm.at[i1_v], r1_v, sem1)
    d0.wait()
    d1.wait()

    def comb_row(r, carry):
        grp = (r >> 4) << 4
        lane = r - grp
        w0g = w0_v[pl.ds(grp, 16)]
        w1g = w1_v[pl.ds(grp, 16)]
        idx = (jnp.zeros((16,), jnp.int32) + lane)[:, None]
        dn = lax.GatherDimensionNumbers(
            offset_dims=(), collapsed_slice_dims=(0,), start_index_map=(0,))
        w0s = lax.gather(w0g, idx, dn, (1,),
                         mode=lax.GatherScatterMode.PROMISE_IN_BOUNDS)
        w1s = lax.gather(w1g, idx, dn, (1,),
                         mode=lax.GatherScatterMode.PROMISE_IN_BOUNDS)
        for c in range(H // 16):
            sl = pl.ds(c * 16, 16)
            r0_v[r, sl] = r0_v[r, sl] * w0s + r1_v[r, sl] * w1s
        return carry

    lax.fori_loop(0, TPT, comb_row, 0)
    pltpu.sync_copy(r0_v, out_hbm.at[pl.ds(tb, TPT), :])


def kernel(x, gate_w, gate_b, W1, b1, W2, b2):
    bsz, seq, hidden = x.shape
    x2d = x.reshape(T, H)
    cidx, wvals, meta, laux, counts = _gating(x2d, gate_w, gate_b)
    cidx3 = cidx.reshape(2, NW, TPT).transpose(1, 0, 2)
    wv3 = wvals.reshape(2, NW, TPT).transpose(1, 0, 2)
    xs = _dispatch(x2d, cidx3)
    ffn_out = _gffn(meta[0], meta[1], meta[2], xs, W1, b1, W2, b2)
    out2d = _combine(ffn_out, cidx3, wv3)
    return out2d.reshape(bsz, seq, hidden), laux[0, 0], counts.reshape(E)
